# baseline (speedup 1.0000x reference)
import functools

import jax
import jax.numpy as jnp
from jax import lax
from jax.experimental import pallas as pl
from jax.experimental.pallas import tpu as pltpu
from jax.experimental.pallas import tpu_sc as plsc

_NC = 2
_NS = 16
_NW = _NC * _NS

_PER_W = 13312


def _take_sc(flat_t, idx, n_rows):
    mesh = plsc.VectorSubcoreMesh(core_axis_name="c", subcore_axis_name="s")
    n_chunks = 4
    chunk = _PER_W // n_chunks

    @functools.partial(
        pl.kernel,
        mesh=mesh,
        out_type=jax.ShapeDtypeStruct((_NW, _PER_W), jnp.float32),
        scratch_types=[
            pltpu.VMEM((_PER_W,), jnp.int32),
            pltpu.VMEM((_PER_W,), jnp.float32),
            pltpu.SemaphoreType.DMA,
        ],
    )
    def k(flat_hbm, idx_hbm, out_hbm, idx_v, vals_v, sem):
        wid = lax.axis_index("s") * _NC + lax.axis_index("c")
        pltpu.sync_copy(idx_hbm.at[wid], idx_v)

        def remap(j, carry):
            v = idx_v[pl.ds(j * 16, 16)]
            r = lax.shift_right_logical(v, 6)
            c = jnp.bitwise_and(v, 63)
            p = (
                lax.shift_right_logical(c, 3) * (n_rows * 8)
                + lax.shift_right_logical(r, 7) * 1024
                + jnp.bitwise_and(c, 7) * 128
                + jnp.bitwise_and(r, 127)
            )
            idx_v[pl.ds(j * 16, 16)] = p
            return carry

        copies = []
        for ci in range(n_chunks):
            lo = ci * chunk
            lax.fori_loop(lo // 16, (lo + chunk) // 16, remap, 0, unroll=4)
            copies.append(
                pltpu.make_async_copy(
                    flat_hbm.at[idx_v.at[pl.ds(lo, chunk)]],
                    vals_v.at[pl.ds(lo, chunk)],
                    sem,
                )
            )
            copies[-1].start()
        for cp in copies:
            cp.wait()
        pltpu.sync_copy(vals_v, out_hbm.at[wid])

    return k(flat_t, idx)


def _widen_tc(xt, tail_t, rp):
    n = xt.shape[1]
    n_main = (n // 128) * 128
    n_tail = n - n_main

    def body(x_ref, tail_ref, out_ref, stage_small, stage_tile, sem1, sem2, sem3):
        cp1 = pltpu.make_async_copy(
            x_ref.at[:, pl.ds(0, n_main)], out_ref.at[:, pl.ds(0, n_main)], sem1
        )
        cp1.start()
        cp2 = pltpu.make_async_copy(tail_ref, stage_small, sem2)
        cp2.start()
        cp2.wait()
        stage_tile[:, 0:n_tail] = stage_small[...]
        cp3 = pltpu.make_async_copy(
            stage_tile, out_ref.at[:, pl.ds(n_main, 128)], sem3
        )
        cp3.start()
        cp3.wait()
        cp1.wait()

    return pl.pallas_call(
        body,
        in_specs=[
            pl.BlockSpec(memory_space=pl.ANY),
            pl.BlockSpec(memory_space=pl.ANY),
        ],
        out_specs=pl.BlockSpec(memory_space=pl.ANY),
        out_shape=jax.ShapeDtypeStruct((xt.shape[0], rp), xt.dtype),
        scratch_shapes=[
            pltpu.VMEM((xt.shape[0], n_tail), xt.dtype),
            pltpu.VMEM((xt.shape[0], 128), xt.dtype),
            pltpu.SemaphoreType.DMA,
            pltpu.SemaphoreType.DMA,
            pltpu.SemaphoreType.DMA,
        ],
    )(xt, tail_t)


def kernel(x, index, out):
    n_pad = -x.shape[0] % 128
    rp = x.shape[0] + n_pad
    xt = jnp.transpose(x)
    n_main = (x.shape[0] // 128) * 128
    xpt = _widen_tc(xt, xt[:, n_main:], rp)
    flat_t = (
        xpt
        .reshape(8, 8, rp // 128, 128)
        .transpose(0, 2, 1, 3)
        .reshape(-1)
    )
    idx_t = jnp.transpose(index).astype(jnp.int32).reshape(_NW, _PER_W)
    gathered = _take_sc(flat_t, idx_t, rp)
    return jnp.transpose(gathered.reshape(index.shape[1], index.shape[0]))


# speedup vs baseline: 13.8964x; 13.8964x over previous
import functools

import jax
import jax.numpy as jnp
from jax import lax
from jax.experimental import pallas as pl
from jax.experimental.pallas import tpu as pltpu
from jax.experimental.pallas import tpu_sc as plsc

_NC = 2
_NS = 16
_NW = _NC * _NS

_PER_W = 13312


def _take_sc(flat_t, idx):
    mesh = plsc.VectorSubcoreMesh(core_axis_name="c", subcore_axis_name="s")

    @functools.partial(
        pl.kernel,
        mesh=mesh,
        out_type=jax.ShapeDtypeStruct((_NW, _PER_W), jnp.float32),
        scratch_types=[
            pltpu.VMEM((_PER_W,), jnp.int32),
            pltpu.VMEM((_PER_W,), jnp.float32),
            pltpu.SemaphoreType.DMA,
        ],
    )
    def k(flat_hbm, idx_hbm, out_hbm, idx_v, vals_v, sem):
        wid = lax.axis_index("s") * _NC + lax.axis_index("c")
        pltpu.sync_copy(idx_hbm.at[wid], idx_v)
        pltpu.async_copy(flat_hbm.at[idx_v], vals_v, sem).wait()
        pltpu.sync_copy(vals_v, out_hbm.at[wid])

    return k(flat_t, idx)


def kernel(x, index, out):
    n_pad = -x.shape[0] % 128
    rp = x.shape[0] + n_pad
    xp = jnp.pad(x, ((0, n_pad), (0, 0)))
    flat_t = (
        jnp.transpose(xp)
        .reshape(8, 8, rp // 128, 128)
        .transpose(0, 2, 1, 3)
        .reshape(-1)
    )
    v = index.astype(jnp.int32)
    r = jnp.right_shift(v, 6)
    c = jnp.bitwise_and(v, 63)
    p = (
        jnp.right_shift(c, 3) * (rp * 8)
        + jnp.right_shift(r, 7) * 1024
        + jnp.bitwise_and(c, 7) * 128
        + jnp.bitwise_and(r, 127)
    )
    idx_t = jnp.transpose(p).reshape(_NW, _PER_W)
    gathered = _take_sc(flat_t, idx_t)
    return jnp.transpose(gathered.reshape(index.shape[1], index.shape[0]))
